# Initial kernel scaffold; baseline (speedup 1.0000x reference)
import jax
import jax.numpy as jnp
from jax.experimental import pallas as pl


def kernel(vectors, centroids, assignment):
    raise NotImplementedError("write your pallas kernel here")


# speedup vs baseline: 3.0489x; 3.0489x over previous
import functools

import jax
import jax.numpy as jnp
from jax import lax
from jax.experimental import pallas as pl
from jax.experimental.pallas import tpu as pltpu
from jax.experimental.pallas import tpu_sc as plsc

_N = 65536
_K = 512
_W = 16
_NC = 2
_NS = 16
_NW = _NC * _NS
_PPW = _N // _NW
_CHUNK = 128
_NCHUNK = _PPW // _CHUNK

_BP = 2048
_NB = _N // _BP


_sc_mesh = plsc.VectorSubcoreMesh(core_axis_name="c", subcore_axis_name="s")


@functools.partial(
    pl.kernel,
    out_type=jax.ShapeDtypeStruct((_NC, _K, _W), jnp.float32),
    mesh=_sc_mesh,
    scratch_types=[
        pltpu.VMEM((_PPW, _W), jnp.float32),
        pltpu.VMEM((_NCHUNK, _CHUNK), jnp.int32),
        pltpu.VMEM_SHARED((_K, _W), jnp.float32),
    ],
    compiler_params=pltpu.CompilerParams(use_tc_tiling_on_sc=False),
)
def _sc_segsum(aug_hbm, idx_hbm, zero_hbm, out_hbm, aug_v, idx_v, acc_sh):
    c = lax.axis_index("c")
    s = lax.axis_index("s")
    w = c * _NS + s
    pltpu.sync_copy(idx_hbm.at[w], idx_v)
    pltpu.sync_copy(aug_hbm.at[pl.ds(w * _PPW, _PPW)], aug_v)
    @pl.when(s == 0)
    def _():
        pltpu.sync_copy(zero_hbm, acc_sh)
    plsc.subcore_barrier()
    for j in range(_NCHUNK):
        pltpu.sync_copy(
            aug_v.at[pl.ds(j * _CHUNK, _CHUNK)],
            acc_sh.at[idx_v.at[j]],
            add=True,
        )
    plsc.subcore_barrier()
    @pl.when(s == 0)
    def _():
        pltpu.sync_copy(acc_sh, out_hbm.at[c])


def _tc_body(partials_ref, vt_ref, cent_ref, assign_ref):
    i = pl.program_id(0)
    psum = partials_ref[0, :, :] + partials_ref[1, :, :]
    cnt = psum[:, 2:3]
    cx = psum[:, 0:1] / cnt
    cy = psum[:, 1:2] / cnt

    @pl.when(i == 0)
    def _():
        cent_ref[...] = jnp.concatenate([cx, cy], axis=1)

    vx = vt_ref[0:1, :]
    vy = vt_ref[1:2, :]
    dx = vx - cx
    dy = vy - cy
    dist = dx * dx + dy * dy
    m = jnp.min(dist, axis=0, keepdims=True)
    rid = lax.broadcasted_iota(jnp.int32, dist.shape, 0)
    sel = jnp.where(dist == m, rid, _K)
    assign_ref[...] = jnp.min(sel, axis=0, keepdims=True)


_tc_assign = pl.pallas_call(
    _tc_body,
    grid=(_NB,),
    in_specs=[
        pl.BlockSpec((_NC, _K, _W), lambda i: (0, 0, 0)),
        pl.BlockSpec((2, _BP), lambda i: (0, i)),
    ],
    out_specs=[
        pl.BlockSpec((_K, 2), lambda i: (0, 0)),
        pl.BlockSpec((1, _BP), lambda i: (0, i)),
    ],
    out_shape=[
        jax.ShapeDtypeStruct((_K, 2), jnp.float32),
        jax.ShapeDtypeStruct((1, _N), jnp.int32),
    ],
)


def kernel(vectors, centroids, assignment):
    del centroids
    aug = jnp.concatenate(
        [
            vectors,
            jnp.ones((_N, 1), jnp.float32),
            jnp.zeros((_N, _W - 3), jnp.float32),
        ],
        axis=1,
    )
    idx3 = assignment.reshape(_NW, _NCHUNK, _CHUNK)
    zero = jnp.zeros((_K, _W), jnp.float32)
    partials = _sc_segsum(aug, idx3, zero)
    cent, a2 = _tc_assign(partials, vectors.T)
    return cent, a2.reshape(_N)


# speedup vs baseline: 3.3183x; 1.0884x over previous
import functools

import jax
import jax.numpy as jnp
from jax import lax
from jax.experimental import pallas as pl
from jax.experimental.pallas import tpu as pltpu
from jax.experimental.pallas import tpu_sc as plsc

_N = 65536
_K = 512
_W = 16
_NC = 2
_NS = 16
_NW = _NC * _NS
_PPW = _N // _NW
_CHUNK = 128
_NCHUNK = _PPW // _CHUNK
_L = 16

_BP = 2048
_NB = _N // _BP


_sc_mesh = plsc.VectorSubcoreMesh(core_axis_name="c", subcore_axis_name="s")


@functools.partial(
    pl.kernel,
    out_type=(
        jax.ShapeDtypeStruct((_NC, _K, _W), jnp.float32),
        jax.ShapeDtypeStruct((2, _N), jnp.float32),
    ),
    mesh=_sc_mesh,
    scratch_types=[
        pltpu.VMEM((2 * _PPW,), jnp.float32),
        pltpu.VMEM((_PPW, _W), jnp.float32),
        pltpu.VMEM((2, _PPW), jnp.float32),
        pltpu.VMEM((_NCHUNK, _CHUNK), jnp.int32),
        pltpu.VMEM_SHARED((_K, _W), jnp.float32),
    ],
    compiler_params=pltpu.CompilerParams(
        use_tc_tiling_on_sc=False, needs_layout_passes=False
    ),
)
def _sc_segsum(vec_hbm, idx_hbm, zero_hbm, part_hbm, vt_hbm,
               raw_v, aug_v, vt_v, idx_v, acc_sh):
    c = lax.axis_index("c")
    s = lax.axis_index("s")
    w = c * _NS + s
    pltpu.sync_copy(idx_hbm.at[w], idx_v)
    pltpu.sync_copy(vec_hbm.at[pl.ds(w * 2 * _PPW, 2 * _PPW)], raw_v)
    @pl.when(s == 0)
    def _():
        pltpu.sync_copy(zero_hbm, acc_sh)

    ids = lax.iota(jnp.int32, _L)
    coord = lax.bitwise_and(ids, 1)
    half = lax.shift_right_logical(ids, 1)
    ones = jnp.full((_L,), 1.0, jnp.float32)

    def deint_body(i, carry):
        v = raw_v[pl.ds(i * _L, _L)]
        row = half + i * (_L // 2)
        plsc.store_scatter(aug_v, [row, coord], v)
        plsc.store_scatter(vt_v, [coord, row], v)
        return carry

    lax.fori_loop(0, 2 * _PPW // _L, deint_body, 0)

    def ones_body(i, carry):
        plsc.store_scatter(aug_v, [ids + i * _L, jnp.full((_L,), 2, jnp.int32)], ones)
        return carry

    lax.fori_loop(0, _PPW // _L, ones_body, 0)

    pltpu.sync_copy(vt_v.at[0], vt_hbm.at[0, pl.ds(w * _PPW, _PPW)])
    pltpu.sync_copy(vt_v.at[1], vt_hbm.at[1, pl.ds(w * _PPW, _PPW)])

    plsc.subcore_barrier()
    for j in range(_NCHUNK):
        pltpu.sync_copy(
            aug_v.at[pl.ds(j * _CHUNK, _CHUNK)],
            acc_sh.at[idx_v.at[j]],
            add=True,
        )
    plsc.subcore_barrier()
    @pl.when(s == 0)
    def _():
        pltpu.sync_copy(acc_sh, part_hbm.at[c])


_R = 8
_NR = _K // _R


def _tc_body(partials_ref, vt_ref, cent_ref, assign_ref):
    i = pl.program_id(0)
    psum = partials_ref[0, :, :] + partials_ref[1, :, :]
    cnt = psum[:, 2:3]
    cx = psum[:, 0:1] / cnt
    cy = psum[:, 1:2] / cnt

    @pl.when(i == 0)
    def _():
        cent_ref[...] = jnp.concatenate([cx, cy], axis=1)

    vx = vt_ref[0:1, :]
    vy = vt_ref[1:2, :]
    riota = lax.broadcasted_iota(jnp.int32, (_R, 1), 0).astype(jnp.float32)
    m = jnp.full((_R, _BP), jnp.inf, jnp.float32)
    bi = jnp.zeros((_R, _BP), jnp.float32)
    for j in range(_NR):
        cxj = cx[j * _R:(j + 1) * _R, :]
        cyj = cy[j * _R:(j + 1) * _R, :]
        dx = vx - cxj
        dy = vy - cyj
        d = dx * dx + dy * dy
        better = d < m
        m = jnp.where(better, d, m)
        bi = jnp.where(better, riota + jnp.float32(j * _R), bi)
    m_all = jnp.min(m, axis=0, keepdims=True)
    idx = jnp.min(jnp.where(m == m_all, bi, jnp.float32(_K)), axis=0, keepdims=True)
    assign_ref[...] = idx.astype(jnp.int32)


_tc_assign = pl.pallas_call(
    _tc_body,
    grid=(_NB,),
    in_specs=[
        pl.BlockSpec((_NC, _K, _W), lambda i: (0, 0, 0)),
        pl.BlockSpec((2, _BP), lambda i: (0, i)),
    ],
    out_specs=[
        pl.BlockSpec((_K, 2), lambda i: (0, 0)),
        pl.BlockSpec((1, _BP), lambda i: (0, i)),
    ],
    out_shape=[
        jax.ShapeDtypeStruct((_K, 2), jnp.float32),
        jax.ShapeDtypeStruct((1, _N), jnp.int32),
    ],
)


def kernel(vectors, centroids, assignment):
    del centroids
    vflat = vectors.reshape(2 * _N)
    idx3 = assignment.reshape(_NW, _NCHUNK, _CHUNK)
    zero = jnp.zeros((_K, _W), jnp.float32)
    partials, vt = _sc_segsum(vflat, idx3, zero)
    cent, a2 = _tc_assign(partials, vt)
    return cent, a2.reshape(_N)
